# baseline (speedup 1.0000x reference)
import functools

import jax
import jax.numpy as jnp
import numpy as np
from jax import lax
from jax.experimental import pallas as pl
from jax.experimental.pallas import tpu as pltpu

N = 10000
E = 160000
E2 = 320000

BE = 640
BN = 1000

_F32 = jnp.float32

_DIV = np.exp(np.log(10000.0) * (2.0 * np.arange(8, dtype=np.float32) / 16.0))
_ANG_SCALE = (8.0 / _DIV).astype(np.float32)
_PE_PERM = np.concatenate([np.arange(0, 16, 2), np.arange(1, 16, 2)])


def _silu(v):
    return v * jax.nn.sigmoid(v)


def _row_spec(b, w):
    return pl.BlockSpec((b, w), lambda i: (i, 0))


def _full_spec(shape):
    nd = len(shape)
    return pl.BlockSpec(shape, lambda i: (0,) * nd)


def _efn_body(hr, hc, w1r, w1c, b1, w2, b2, out):
    t = (jnp.dot(hr[...], w1r[...], preferred_element_type=_F32)
         + jnp.dot(hc[...], w1c[...], preferred_element_type=_F32) + b1[...])
    t = _silu(t)
    out[...] = jnp.dot(t, w2[...], preferred_element_type=_F32) + b2[...]


def _tc_efn(hr, hc, w1r, w1c, b1, w2, b2):
    grid = (E // BE,)
    return pl.pallas_call(
        _efn_body,
        grid=grid,
        in_specs=[_row_spec(BE, 128), _row_spec(BE, 128),
                  _full_spec((128, 128)), _full_spec((128, 128)),
                  _full_spec((1, 128)), _full_spec((128, 128)),
                  _full_spec((1, 128))],
        out_specs=_row_spec(BE, 128),
        out_shape=jax.ShapeDtypeStruct((E, 128), _F32),
    )(hr, hc, w1r, w1c, b1, w2, b2)


def _m1m2_body(prod, ff, angs, p1pe, p1ff, p1b, p2pe, p2ff, p2b,
               sincos_out, m1_out, m2_out):
    vtv = jnp.sum(prod[...], axis=1, keepdims=True)
    ang = vtv * angs[...]
    sincos = jnp.concatenate([jnp.sin(ang), jnp.cos(ang)], axis=1)
    sincos_out[...] = sincos
    ffv = ff[...]
    m1 = (jnp.dot(sincos, p1pe[...], preferred_element_type=_F32)
          + jnp.dot(ffv, p1ff[...], preferred_element_type=_F32) + p1b[...])
    m1_out[...] = _silu(m1)
    m2 = (jnp.dot(sincos, p2pe[...], preferred_element_type=_F32)
          + jnp.dot(ffv, p2ff[...], preferred_element_type=_F32) + p2b[...])
    m2_out[...] = _silu(m2)


def _tc_m1m2(prod, ff, angs, p1pe, p1ff, p1b, p2pe, p2ff, p2b):
    grid = (E2 // BE,)
    return pl.pallas_call(
        _m1m2_body,
        grid=grid,
        in_specs=[_row_spec(BE, 16), _row_spec(BE, 128), _full_spec((1, 8)),
                  _full_spec((16, 128)), _full_spec((128, 128)),
                  _full_spec((1, 128)),
                  _full_spec((16, 128)), _full_spec((128, 128)),
                  _full_spec((1, 128))],
        out_specs=[_row_spec(BE, 16), _row_spec(BE, 128), _row_spec(BE, 128)],
        out_shape=[jax.ShapeDtypeStruct((E2, 16), _F32),
                   jax.ShapeDtypeStruct((E2, 128), _F32),
                   jax.ShapeDtypeStruct((E2, 128), _F32)],
    )(prod, ff, angs, p1pe, p1ff, p1b, p2pe, p2ff, p2b)


def _nbx2_body(sincos, ff, mm, p3pe, p3ff, p3mm, p3b, out):
    out[...] = (jnp.dot(sincos[...], p3pe[...], preferred_element_type=_F32)
                + jnp.dot(ff[...], p3ff[...], preferred_element_type=_F32)
                + jnp.dot(mm[...], p3mm[...], preferred_element_type=_F32)
                + p3b[...])


def _tc_nbx2(sincos, ff, mm, p3pe, p3ff, p3mm, p3b):
    grid = (E2 // BE,)
    return pl.pallas_call(
        _nbx2_body,
        grid=grid,
        in_specs=[_row_spec(BE, 16), _row_spec(BE, 128), _row_spec(BE, 128),
                  _full_spec((16, 128)), _full_spec((128, 128)),
                  _full_spec((128, 128)), _full_spec((1, 128))],
        out_specs=_row_spec(BE, 128),
        out_shape=jax.ShapeDtypeStruct((E2, 128), _F32),
    )(sincos, ff, mm, p3pe, p3ff, p3mm, p3b)


def _t1_body(sr, sc, cd, ign_r, ign_c, ignb, cw1, cb1, cw2, one3,
             t1_out, trans_out):
    t1 = (jnp.dot(sr[...], ign_r[...], preferred_element_type=_F32)
          + jnp.dot(sc[...], ign_c[...], preferred_element_type=_F32)
          + ignb[...])
    t1_out[...] = t1
    u = _silu(jnp.dot(t1, cw1[...], preferred_element_type=_F32) + cb1[...])
    w = jnp.dot(u, cw2[...], preferred_element_type=_F32)
    trans_out[...] = cd[...] * w + one3[...]


def _tc_t1(sr, sc, cd, ign_r, ign_c, ignb, cw1, cb1, cw2, one3):
    grid = (E // BE,)
    return pl.pallas_call(
        _t1_body,
        grid=grid,
        in_specs=[_row_spec(BE, 128), _row_spec(BE, 128), _row_spec(BE, 16),
                  _full_spec((128, 128)), _full_spec((128, 128)),
                  _full_spec((1, 128)), _full_spec((128, 128)),
                  _full_spec((1, 128)), _full_spec((128, 1)),
                  _full_spec((1, 16))],
        out_specs=[_row_spec(BE, 128), _row_spec(BE, 16)],
        out_shape=[jax.ShapeDtypeStruct((E, 128), _F32),
                   jax.ShapeDtypeStruct((E, 16), _F32)],
    )(sr, sc, cd, ign_r, ign_c, ignb, cw1, cb1, cw2, one3)


def _final_body(x16, aggc, t0, h, nw1, nb1, nw2, nb2, h_out, x_out):
    cnt = jnp.maximum(aggc[...][:, 3:4], 1.0)
    x_out[...] = x16[...] + aggc[...] * (1.0 / cnt)
    u = _silu(jnp.dot(t0[...], nw1[...], preferred_element_type=_F32)
              + nb1[...])
    h_out[...] = h[...] + jnp.dot(u, nw2[...],
                                  preferred_element_type=_F32) + nb2[...]


def _tc_final(x16, aggc, t0, h, nw1, nb1, nw2, nb2):
    grid = (N // BN,)
    return pl.pallas_call(
        _final_body,
        grid=grid,
        in_specs=[_row_spec(BN, 16), _row_spec(BN, 16), _row_spec(BN, 128),
                  _row_spec(BN, 128),
                  _full_spec((128, 128)), _full_spec((1, 128)),
                  _full_spec((128, 128)), _full_spec((1, 128))],
        out_specs=[_row_spec(BN, 128), _row_spec(BN, 16)],
        out_shape=[jax.ShapeDtypeStruct((N, 128), _F32),
                   jax.ShapeDtypeStruct((N, 16), _F32)],
    )(x16, aggc, t0, h, nw1, nb1, nw2, nb2)


def _gather_rows(table, idx):
    return jnp.take(table, idx, axis=0)


def _segsum(vals, idx, num):
    return jax.ops.segment_sum(vals, idx, num_segments=num)


def kernel(h, x, edges, nb_edge, edge_attr, nb_num_nodes, params):
    del nb_num_nodes
    rows, cols = edges[0], edges[1]
    nbr, nbc = nb_edge[0], nb_edge[1]

    x16 = jnp.pad(x, ((0, 0), (0, 13)))

    w1r = params['ee_W1'][:128]
    w1c = params['ee_W1'][128:]
    b1 = params['ee_b1'].reshape(1, 128)
    w2 = params['ee_W2']
    b2 = params['ee_b2'].reshape(1, 128)
    p1pe = params['p1_W'][:16][_PE_PERM]
    p1ff = params['p1_W'][16:]
    p1b = params['p1_b'].reshape(1, 128)
    p2pe = params['p2_W'][:16][_PE_PERM]
    p2ff = params['p2_W'][16:]
    p2b = params['p2_b'].reshape(1, 128)
    p3pe = params['p3_W'][:16][_PE_PERM]
    p3ff = params['p3_W'][16:144]
    p3mm = params['p3_W'][144:]
    p3b = params['p3_b'].reshape(1, 128)
    ign_r = params['ign_W'][:128]
    ign_c = params['ign_W'][128:]
    ignb = params['ign_b'].reshape(1, 128)
    cw1 = params['cm_W1']
    cb1 = params['cm_b1'].reshape(1, 128)
    cw2 = params['cm_W2']
    nw1 = params['nd_W1']
    nb1 = params['nd_b1'].reshape(1, 128)
    nw2 = params['nd_W2']
    nb2 = params['nd_b2'].reshape(1, 128)
    one3 = jnp.zeros((1, 16), _F32).at[0, 3].set(1.0)
    angs = jnp.asarray(_ANG_SCALE).reshape(1, 8)

    cd = _gather_rows(x16, rows) - _gather_rows(x16, cols)
    prod = _gather_rows(cd, nbr) * _gather_rows(cd, nbc)
    hr = _gather_rows(h, rows)
    hc = _gather_rows(h, cols)
    efn = _tc_efn(hr, hc, w1r, w1c, b1, w2, b2)
    ff = _gather_rows(efn, nbr) * _gather_rows(efn, nbc)
    sincos, m1, m2 = _tc_m1m2(prod, ff, angs, p1pe, p1ff, p1b,
                              p2pe, p2ff, p2b)
    s1 = _segsum(m1, nbr, E)
    s2 = _segsum(m2, nbc, E)
    mm = _gather_rows(s1, nbr) * _gather_rows(s2, nbc)
    nb_x2 = _tc_nbx2(sincos, ff, mm, p3pe, p3ff, p3mm, p3b)
    sr = _segsum(nb_x2, nbr, E)
    sc = _segsum(nb_x2, nbc, E)
    t1, trans16 = _tc_t1(sr, sc, cd, ign_r, ign_c, ignb, cw1, cb1, cw2, one3)
    aggc = _segsum(trans16, rows, N)
    t0 = _segsum(t1, rows, N)
    h_out, x16_out = _tc_final(x16, aggc, t0, h, nw1, nb1, nw2, nb2)
    return (h_out, x16_out[:, :3], edge_attr)
